# Initial kernel scaffold; baseline (speedup 1.0000x reference)
import jax
import jax.numpy as jnp
from jax.experimental import pallas as pl

_N1 = 10000
_N2 = 10000
_D = 64
_K = 3
_N1P = 10240
_N2P = 10240
_BQ = 128
_PAD_COORD = 1.0e6
_BIG = jnp.float32(3.0e38)


def _warp_body(c_ref, flow_ref, out_ref):
    out_ref[...] = c_ref[...] + flow_ref[...]


def _knn_body(q_ref, srcT_ref, f_ref, out_ref):
    q = q_ref[...]
    srcT = srcT_ref[...]

    sx = srcT[0:1, :]
    sy = srcT[1:2, :]
    sz = srcT[2:3, :]
    src_sq = (sx * sx + sy * sy) + sz * sz

    qx = q[:, 0:1]
    qy = q[:, 1:2]
    qz = q[:, 2:3]
    q_sq = (qx * qx + qy * qy) + qz * qz

    dot = jnp.dot(q, srcT, preferred_element_type=jnp.float32)
    d2 = (q_sq + src_sq) - 2.0 * dot

    lane = jax.lax.broadcasted_iota(jnp.int32, (_BQ, _N1P), 1)

    idxs = []
    for _ in range(_K):
        minv = jnp.min(d2, axis=1, keepdims=True)
        eq = d2 == minv
        idx = jnp.min(jnp.where(eq, lane, _N1P), axis=1, keepdims=True)
        idxs.append(idx)
        d2 = jnp.where(lane == idx, _BIG, d2)

    rs = []
    for idx in idxs:
        onehot = lane == idx
        gx = jnp.sum(jnp.where(onehot, sx, 0.0), axis=1, keepdims=True)
        gy = jnp.sum(jnp.where(onehot, sy, 0.0), axis=1, keepdims=True)
        gz = jnp.sum(jnp.where(onehot, sz, 0.0), axis=1, keepdims=True)
        dx = gx - qx
        dy = gy - qy
        dz = gz - qz
        dist = jnp.sqrt((dx * dx + dy * dy) + dz * dz)
        dist = jnp.maximum(dist, jnp.float32(1e-10))
        rs.append(1.0 / dist)
    norm = (rs[0] + rs[1]) + rs[2]

    w = jnp.zeros((_BQ, _N1P), dtype=jnp.float32)
    for idx, r in zip(idxs, rs):
        w = jnp.where(lane == idx, r / norm, w)

    out_ref[...] = jnp.dot(w, f_ref[...], preferred_element_type=jnp.float32)


def kernel(f_ref_C, f_ref_F, f_cur_C, point_flow):
    f_ref_warp = pl.pallas_call(
        _warp_body,
        grid=(5,),
        in_specs=[
            pl.BlockSpec((_N1 // 5, 3), lambda i: (i, 0)),
            pl.BlockSpec((_N1 // 5, 3), lambda i: (i, 0)),
        ],
        out_specs=pl.BlockSpec((_N1 // 5, 3), lambda i: (i, 0)),
        out_shape=jax.ShapeDtypeStruct((_N1, 3), jnp.float32),
    )(f_ref_C, point_flow)

    srcT = jnp.pad(f_ref_warp, ((0, _N1P - _N1), (0, 0)),
                   constant_values=_PAD_COORD).T
    f_pad = jnp.pad(f_ref_F, ((0, _N1P - _N1), (0, 0)))
    q_pad = jnp.pad(f_cur_C, ((0, _N2P - _N2), (0, 0)))

    out = pl.pallas_call(
        _knn_body,
        grid=(_N2P // _BQ,),
        in_specs=[
            pl.BlockSpec((_BQ, 3), lambda i: (i, 0)),
            pl.BlockSpec((3, _N1P), lambda i: (0, 0)),
            pl.BlockSpec((_N1P, _D), lambda i: (0, 0)),
        ],
        out_specs=pl.BlockSpec((_BQ, _D), lambda i: (i, 0)),
        out_shape=jax.ShapeDtypeStruct((_N2P, _D), jnp.float32),
    )(q_pad, srcT, f_pad)

    return out[:_N2], f_ref_warp


# speedup vs baseline: 3.5618x; 3.5618x over previous
import jax
import jax.numpy as jnp
from jax.experimental import pallas as pl

_N1 = 10000
_N2 = 10000
_D = 64
_K = 3
_N1P = 10240
_N2P = 10240
_BQ = 128
_PAD_COORD = 1.0e6
_BIG = 3.0e38


def _warp_body(c_ref, flow_ref, out_ref):
    out_ref[...] = c_ref[...] + flow_ref[...]


def _knn_body(q_ref, srcT_ref, f_ref, out_ref):
    q = q_ref[...]
    srcT = srcT_ref[...]

    sx = srcT[0:1, :]
    sy = srcT[1:2, :]
    sz = srcT[2:3, :]
    src_sq = (sx * sx + sy * sy) + sz * sz

    qx = q[:, 0:1]
    qy = q[:, 1:2]
    qz = q[:, 2:3]
    q_sq = (qx * qx + qy * qy) + qz * qz

    dot = jnp.dot(q, srcT, preferred_element_type=jnp.float32)
    d2 = (q_sq + src_sq) - 2.0 * dot

    lane = jax.lax.broadcasted_iota(jnp.int32, (_BQ, _N1P), 1)

    idxs = []
    for _ in range(_K):
        minv = jnp.min(d2, axis=1, keepdims=True)
        eq = d2 == minv
        idx = jnp.min(jnp.where(eq, lane, _N1P), axis=1, keepdims=True)
        idxs.append(idx)
        d2 = jnp.where(lane == idx, _BIG, d2)

    rs = []
    for idx in idxs:
        onehot = lane == idx
        gx = jnp.sum(jnp.where(onehot, sx, 0.0), axis=1, keepdims=True)
        gy = jnp.sum(jnp.where(onehot, sy, 0.0), axis=1, keepdims=True)
        gz = jnp.sum(jnp.where(onehot, sz, 0.0), axis=1, keepdims=True)
        dx = gx - qx
        dy = gy - qy
        dz = gz - qz
        dist = jnp.sqrt((dx * dx + dy * dy) + dz * dz)
        dist = jnp.maximum(dist, 1e-10)
        rs.append(1.0 / dist)
    norm = (rs[0] + rs[1]) + rs[2]

    w = jnp.zeros((_BQ, _N1P), dtype=jnp.float32)
    for idx, r in zip(idxs, rs):
        w = jnp.where(lane == idx, r / norm, w)

    out_ref[...] = jnp.dot(w, f_ref[...], preferred_element_type=jnp.float32)


def kernel(f_ref_C, f_ref_F, f_cur_C, point_flow):
    f_ref_warp = pl.pallas_call(
        _warp_body,
        grid=(5,),
        in_specs=[
            pl.BlockSpec((_N1 // 5, 3), lambda i: (i, 0)),
            pl.BlockSpec((_N1 // 5, 3), lambda i: (i, 0)),
        ],
        out_specs=pl.BlockSpec((_N1 // 5, 3), lambda i: (i, 0)),
        out_shape=jax.ShapeDtypeStruct((_N1, 3), jnp.float32),
    )(f_ref_C, point_flow)

    srcT = jnp.pad(f_ref_warp, ((0, _N1P - _N1), (0, 0)),
                   constant_values=_PAD_COORD).T
    f_pad = jnp.pad(f_ref_F, ((0, _N1P - _N1), (0, 0)))
    q_pad = jnp.pad(f_cur_C, ((0, _N2P - _N2), (0, 0)))

    out = pl.pallas_call(
        _knn_body,
        grid=(_N2P // _BQ,),
        in_specs=[
            pl.BlockSpec((_BQ, 3), lambda i: (i, 0)),
            pl.BlockSpec((3, _N1P), lambda i: (0, 0)),
            pl.BlockSpec((_N1P, _D), lambda i: (0, 0)),
        ],
        out_specs=pl.BlockSpec((_BQ, _D), lambda i: (i, 0)),
        out_shape=jax.ShapeDtypeStruct((_N2P, _D), jnp.float32),
    )(q_pad, srcT, f_pad)

    return out[:_N2], f_ref_warp


# speedup vs baseline: 4.4699x; 1.2550x over previous
import jax
import jax.numpy as jnp
from jax.experimental import pallas as pl

_N1 = 10000
_N2 = 10000
_D = 64
_K = 3
_N1P = 10240
_N2P = 10240
_BQ = 128
_PAD_COORD = 1.0e6
_BIG = 3.0e38


def _warp_body(c_ref, flow_ref, out_ref):
    out_ref[...] = c_ref[...] + flow_ref[...]


def _knn_body(q_ref, srcT_ref, f_ref, out_ref):
    q = q_ref[...]
    srcT = srcT_ref[...]

    sx = srcT[0:1, :]
    sy = srcT[1:2, :]
    sz = srcT[2:3, :]
    src_sq = (sx * sx + sy * sy) + sz * sz

    qx = q[:, 0:1]
    qy = q[:, 1:2]
    qz = q[:, 2:3]
    q_sq = (qx * qx + qy * qy) + qz * qz

    dot = jnp.dot(q, srcT, preferred_element_type=jnp.float32)
    d2 = (q_sq + src_sq) - 2.0 * dot

    dx = sx - qx
    dy = sy - qy
    dz = sz - qz
    d2e = (dx * dx + dy * dy) + dz * dz

    lane = jax.lax.broadcasted_iota(jnp.int32, (_BQ, _N1P), 1)

    idxs = []
    rs = []
    for _ in range(_K):
        minv = jnp.min(d2, axis=1, keepdims=True)
        idxsel = jnp.where(d2 == minv, lane, _N1P)
        idx = jnp.min(idxsel, axis=1, keepdims=True)
        oh = idxsel == idx
        d2 = jnp.where(oh, _BIG, d2)
        d2e_k = jnp.sum(jnp.where(oh, d2e, 0.0), axis=1, keepdims=True)
        dist = jnp.maximum(jnp.sqrt(d2e_k), 1e-10)
        idxs.append(idx)
        rs.append(1.0 / dist)
    norm = (rs[0] + rs[1]) + rs[2]

    w = jnp.zeros((_BQ, _N1P), dtype=jnp.float32)
    for idx, r in zip(idxs, rs):
        w = jnp.where(lane == idx, r / norm, w)

    out_ref[...] = jnp.dot(w, f_ref[...], preferred_element_type=jnp.float32)


def kernel(f_ref_C, f_ref_F, f_cur_C, point_flow):
    f_ref_warp = pl.pallas_call(
        _warp_body,
        grid=(5,),
        in_specs=[
            pl.BlockSpec((_N1 // 5, 3), lambda i: (i, 0)),
            pl.BlockSpec((_N1 // 5, 3), lambda i: (i, 0)),
        ],
        out_specs=pl.BlockSpec((_N1 // 5, 3), lambda i: (i, 0)),
        out_shape=jax.ShapeDtypeStruct((_N1, 3), jnp.float32),
    )(f_ref_C, point_flow)

    srcT = jnp.pad(f_ref_warp, ((0, _N1P - _N1), (0, 0)),
                   constant_values=_PAD_COORD).T
    f_pad = jnp.pad(f_ref_F, ((0, _N1P - _N1), (0, 0)))
    q_pad = jnp.pad(f_cur_C, ((0, _N2P - _N2), (0, 0)))

    out = pl.pallas_call(
        _knn_body,
        grid=(_N2P // _BQ,),
        in_specs=[
            pl.BlockSpec((_BQ, 3), lambda i: (i, 0)),
            pl.BlockSpec((3, _N1P), lambda i: (0, 0)),
            pl.BlockSpec((_N1P, _D), lambda i: (0, 0)),
        ],
        out_specs=pl.BlockSpec((_BQ, _D), lambda i: (i, 0)),
        out_shape=jax.ShapeDtypeStruct((_N2P, _D), jnp.float32),
    )(q_pad, srcT, f_pad)

    return out[:_N2], f_ref_warp


# speedup vs baseline: 4.8940x; 1.0949x over previous
import jax
import jax.numpy as jnp
from jax.experimental import pallas as pl

_N1 = 10000
_N2 = 10000
_D = 64
_K = 3
_N1P = 10240
_N2P = 10240
_BQ = 256
_PAD_COORD = 1.0e6
_BIG = 3.0e38


def _warp_body(c_ref, flow_ref, out_ref):
    out_ref[...] = c_ref[...] + flow_ref[...]


def _knn_body(q_ref, srcT_ref, f_ref, out_ref):
    q = q_ref[...]
    srcT = srcT_ref[...]

    sx = srcT[0:1, :]
    sy = srcT[1:2, :]
    sz = srcT[2:3, :]
    src_sq = (sx * sx + sy * sy) + sz * sz

    qx = q[:, 0:1]
    qy = q[:, 1:2]
    qz = q[:, 2:3]
    q_sq = (qx * qx + qy * qy) + qz * qz

    dot = jnp.dot(q, srcT, preferred_element_type=jnp.float32)
    d2 = (q_sq + src_sq) - 2.0 * dot

    dx = sx - qx
    dy = sy - qy
    dz = sz - qz
    d2e = (dx * dx + dy * dy) + dz * dz

    lane = jax.lax.broadcasted_iota(jnp.int32, (_BQ, _N1P), 1)

    idxs = []
    rs = []
    for _ in range(_K):
        idx = jnp.argmin(d2, axis=1)[:, None].astype(jnp.int32)
        oh = lane == idx
        d2 = jnp.where(oh, _BIG, d2)
        d2e_k = jnp.sum(jnp.where(oh, d2e, 0.0), axis=1, keepdims=True)
        dist = jnp.maximum(jnp.sqrt(d2e_k), 1e-10)
        idxs.append(idx)
        rs.append(1.0 / dist)
    norm = (rs[0] + rs[1]) + rs[2]

    w = jnp.zeros((_BQ, _N1P), dtype=jnp.float32)
    for idx, r in zip(idxs, rs):
        w = jnp.where(lane == idx, r / norm, w)

    out_ref[...] = jnp.dot(w, f_ref[...], preferred_element_type=jnp.float32)


def kernel(f_ref_C, f_ref_F, f_cur_C, point_flow):
    f_ref_warp = pl.pallas_call(
        _warp_body,
        grid=(5,),
        in_specs=[
            pl.BlockSpec((_N1 // 5, 3), lambda i: (i, 0)),
            pl.BlockSpec((_N1 // 5, 3), lambda i: (i, 0)),
        ],
        out_specs=pl.BlockSpec((_N1 // 5, 3), lambda i: (i, 0)),
        out_shape=jax.ShapeDtypeStruct((_N1, 3), jnp.float32),
    )(f_ref_C, point_flow)

    srcT = jnp.pad(f_ref_warp, ((0, _N1P - _N1), (0, 0)),
                   constant_values=_PAD_COORD).T
    f_pad = jnp.pad(f_ref_F, ((0, _N1P - _N1), (0, 0)))
    q_pad = jnp.pad(f_cur_C, ((0, _N2P - _N2), (0, 0)))

    out = pl.pallas_call(
        _knn_body,
        grid=(_N2P // _BQ,),
        in_specs=[
            pl.BlockSpec((_BQ, 3), lambda i: (i, 0)),
            pl.BlockSpec((3, _N1P), lambda i: (0, 0)),
            pl.BlockSpec((_N1P, _D), lambda i: (0, 0)),
        ],
        out_specs=pl.BlockSpec((_BQ, _D), lambda i: (i, 0)),
        out_shape=jax.ShapeDtypeStruct((_N2P, _D), jnp.float32),
    )(q_pad, srcT, f_pad)

    return out[:_N2], f_ref_warp


# speedup vs baseline: 5.0587x; 1.0337x over previous
import jax
import jax.numpy as jnp
from jax.experimental import pallas as pl
from jax.experimental.pallas import tpu as pltpu
from jax.experimental.pallas import tpu_sc as plsc

_N1 = 10000
_N2 = 10000
_D = 64
_K = 3
_N1P = 10240
_N2P = 10240
_BQ = 256
_BC = 512
_PAD_COORD = 1.0e6
_BIG = 3.0e38

_DP = 128
_NW = 32
_RPW = _N2P * _K // _NW
_CHUNK = 120
_NCH = _RPW // _CHUNK


def _warp_body(c_ref, flow_ref, out_ref):
    out_ref[...] = c_ref[...] + flow_ref[...]


def _knn_body(q_ref, srcT_ref, idx_ref, w_ref):
    q = q_ref[...]
    srcT = srcT_ref[...]

    sx = srcT[0:1, :]
    sy = srcT[1:2, :]
    sz = srcT[2:3, :]
    src_sq = (sx * sx + sy * sy) + sz * sz

    qx = q[:, 0:1]
    qy = q[:, 1:2]
    qz = q[:, 2:3]
    q_sq = (qx * qx + qy * qy) + qz * qz

    dot = jnp.dot(q, srcT, preferred_element_type=jnp.float32)
    d2 = (q_sq + src_sq) - 2.0 * dot

    dx = sx - qx
    dy = sy - qy
    dz = sz - qz
    d2e = (dx * dx + dy * dy) + dz * dz

    lane = jax.lax.broadcasted_iota(jnp.int32, (_BQ, _N1P), 1)

    idxs = []
    rs = []
    for _ in range(_K):
        idx = jnp.argmin(d2, axis=1)[:, None].astype(jnp.int32)
        oh = lane == idx
        d2 = jnp.where(oh, _BIG, d2)
        d2e_k = jnp.sum(jnp.where(oh, d2e, 0.0), axis=1, keepdims=True)
        dist = jnp.maximum(jnp.sqrt(d2e_k), 1e-10)
        idxs.append(idx)
        rs.append(1.0 / dist)
    norm = (rs[0] + rs[1]) + rs[2]

    idx_ref[...] = jnp.concatenate(idxs, axis=1)
    w_ref[...] = jnp.concatenate([r / norm for r in rs], axis=1)


def _gather_body(table_hbm, idx_hbm, out_hbm, idx_v, rows_v, sem):
    wid = jax.lax.axis_index("s") * 2 + jax.lax.axis_index("c")
    base = wid * _RPW
    for c in range(_NCH):
        off = base + c * _CHUNK
        pltpu.sync_copy(idx_hbm.at[pl.ds(off, _CHUNK)], idx_v)
        pltpu.async_copy(table_hbm.at[idx_v], rows_v, sem).wait()
        pltpu.sync_copy(rows_v, out_hbm.at[pl.ds(off, _CHUNK)])


def _combine_body(g_ref, w_ref, out_ref):
    g = g_ref[...]
    w = w_ref[...]
    out_ref[...] = (g[:, 0:_D] * w[:, 0:1]
                    + g[:, _DP:_DP + _D] * w[:, 1:2]) \
        + g[:, 2 * _DP:2 * _DP + _D] * w[:, 2:3]


def kernel(f_ref_C, f_ref_F, f_cur_C, point_flow):
    f_ref_warp = pl.pallas_call(
        _warp_body,
        grid=(5,),
        in_specs=[
            pl.BlockSpec((_N1 // 5, 3), lambda i: (i, 0)),
            pl.BlockSpec((_N1 // 5, 3), lambda i: (i, 0)),
        ],
        out_specs=pl.BlockSpec((_N1 // 5, 3), lambda i: (i, 0)),
        out_shape=jax.ShapeDtypeStruct((_N1, 3), jnp.float32),
    )(f_ref_C, point_flow)

    srcT = jnp.pad(f_ref_warp, ((0, _N1P - _N1), (0, 0)),
                   constant_values=_PAD_COORD).T
    q_pad = jnp.pad(f_cur_C, ((0, _N2P - _N2), (0, 0)))

    knn_idx, knn_w = pl.pallas_call(
        _knn_body,
        grid=(_N2P // _BQ,),
        in_specs=[
            pl.BlockSpec((_BQ, 3), lambda i: (i, 0)),
            pl.BlockSpec((3, _N1P), lambda i: (0, 0)),
        ],
        out_specs=[
            pl.BlockSpec((_BQ, _K), lambda i: (i, 0)),
            pl.BlockSpec((_BQ, _K), lambda i: (i, 0)),
        ],
        out_shape=[
            jax.ShapeDtypeStruct((_N2P, _K), jnp.int32),
            jax.ShapeDtypeStruct((_N2P, _K), jnp.float32),
        ],
    )(q_pad, srcT)

    idx_flat = knn_idx.reshape(_N2P * _K)
    mesh = plsc.VectorSubcoreMesh(core_axis_name="c", subcore_axis_name="s")
    grouped = pl.kernel(
        _gather_body,
        mesh=mesh,
        out_type=jax.ShapeDtypeStruct((_N2P * _K, _DP), jnp.float32),
        scratch_types=[
            pltpu.VMEM((_CHUNK,), jnp.int32),
            pltpu.VMEM((_CHUNK, _DP), jnp.float32),
            pltpu.SemaphoreType.DMA,
        ],
    )(jnp.pad(f_ref_F, ((0, 0), (0, _DP - _D))), idx_flat)

    out = pl.pallas_call(
        _combine_body,
        grid=(_N2P // _BC,),
        in_specs=[
            pl.BlockSpec((_BC, _K * _DP), lambda i: (i, 0)),
            pl.BlockSpec((_BC, _K), lambda i: (i, 0)),
        ],
        out_specs=pl.BlockSpec((_BC, _D), lambda i: (i, 0)),
        out_shape=jax.ShapeDtypeStruct((_N2P, _D), jnp.float32),
    )(grouped.reshape(_N2P, _K * _DP), knn_w)

    return out[:_N2], f_ref_warp


# speedup vs baseline: 7.9445x; 1.5705x over previous
import jax
import jax.numpy as jnp
from jax.experimental import pallas as pl
from jax.experimental.pallas import tpu as pltpu
from jax.experimental.pallas import tpu_sc as plsc

_N1 = 10000
_N2 = 10000
_D = 64
_K = 3
_N1P = 10240
_N2P = 10240
_BQ = 256
_BC = 512
_PAD_COORD = 1.0e6
_BIG = 3.0e38

_DP = 128
_NW = 32
_RPW = _N2P * _K // _NW
_CHUNK = 120
_NCH = _RPW // _CHUNK


def _warp_body(c_ref, flow_ref, out_ref):
    out_ref[...] = c_ref[...] + flow_ref[...]


def _knn_body(q_ref, srcT_ref, idx_ref):
    q = q_ref[...]
    srcT = srcT_ref[...]

    sx = srcT[0:1, :]
    sy = srcT[1:2, :]
    sz = srcT[2:3, :]
    src_sq = (sx * sx + sy * sy) + sz * sz

    qx = q[:, 0:1]
    qy = q[:, 1:2]
    qz = q[:, 2:3]
    q_sq = (qx * qx + qy * qy) + qz * qz

    dot = jnp.dot(q, srcT, preferred_element_type=jnp.float32)
    d2 = (q_sq + src_sq) - 2.0 * dot

    lane = jax.lax.broadcasted_iota(jnp.int32, (_BQ, _N1P), 1)

    idxs = []
    for _ in range(_K):
        idx = jnp.argmin(d2, axis=1)[:, None].astype(jnp.int32)
        d2 = jnp.where(lane == idx, _BIG, d2)
        idxs.append(idx)

    idx_ref[...] = jnp.concatenate(idxs, axis=1)


def _gather_body(table_hbm, idx_hbm, out_hbm, idx_v, rows_v, sem):
    wid = jax.lax.axis_index("s") * 2 + jax.lax.axis_index("c")
    base = wid * _RPW
    for c in range(_NCH):
        off = base + c * _CHUNK
        pltpu.sync_copy(idx_hbm.at[pl.ds(off, _CHUNK)], idx_v)
        pltpu.async_copy(table_hbm.at[idx_v], rows_v, sem).wait()
        pltpu.sync_copy(rows_v, out_hbm.at[pl.ds(off, _CHUNK)])


def _combine_body(g_ref, q_ref, out_ref):
    g = g_ref[...]
    q = q_ref[...]
    qx = q[:, 0:1]
    qy = q[:, 1:2]
    qz = q[:, 2:3]
    rs = []
    for k in range(_K):
        dx = g[:, k * _DP + _D:k * _DP + _D + 1] - qx
        dy = g[:, k * _DP + _D + 1:k * _DP + _D + 2] - qy
        dz = g[:, k * _DP + _D + 2:k * _DP + _D + 3] - qz
        dist = jnp.sqrt((dx * dx + dy * dy) + dz * dz)
        rs.append(1.0 / jnp.maximum(dist, 1e-10))
    norm = (rs[0] + rs[1]) + rs[2]
    out_ref[...] = (g[:, 0:_D] * (rs[0] / norm)
                    + g[:, _DP:_DP + _D] * (rs[1] / norm)) \
        + g[:, 2 * _DP:2 * _DP + _D] * (rs[2] / norm)


def kernel(f_ref_C, f_ref_F, f_cur_C, point_flow):
    f_ref_warp = pl.pallas_call(
        _warp_body,
        grid=(5,),
        in_specs=[
            pl.BlockSpec((_N1 // 5, 3), lambda i: (i, 0)),
            pl.BlockSpec((_N1 // 5, 3), lambda i: (i, 0)),
        ],
        out_specs=pl.BlockSpec((_N1 // 5, 3), lambda i: (i, 0)),
        out_shape=jax.ShapeDtypeStruct((_N1, 3), jnp.float32),
    )(f_ref_C, point_flow)

    srcT = jnp.pad(f_ref_warp, ((0, _N1P - _N1), (0, 0)),
                   constant_values=_PAD_COORD).T
    q_pad = jnp.pad(f_cur_C, ((0, _N2P - _N2), (0, 0)))
    table = jnp.pad(jnp.concatenate([f_ref_F, f_ref_warp], axis=1),
                    ((0, 0), (0, _DP - _D - 3)))

    knn_idx = pl.pallas_call(
        _knn_body,
        grid=(_N2P // _BQ,),
        in_specs=[
            pl.BlockSpec((_BQ, 3), lambda i: (i, 0)),
            pl.BlockSpec((3, _N1P), lambda i: (0, 0)),
        ],
        out_specs=pl.BlockSpec((_BQ, _K), lambda i: (i, 0)),
        out_shape=jax.ShapeDtypeStruct((_N2P, _K), jnp.int32),
    )(q_pad, srcT)

    idx_flat = knn_idx.reshape(_N2P * _K)
    mesh = plsc.VectorSubcoreMesh(core_axis_name="c", subcore_axis_name="s")
    grouped = pl.kernel(
        _gather_body,
        mesh=mesh,
        out_type=jax.ShapeDtypeStruct((_N2P * _K, _DP), jnp.float32),
        scratch_types=[
            pltpu.VMEM((_CHUNK,), jnp.int32),
            pltpu.VMEM((_CHUNK, _DP), jnp.float32),
            pltpu.SemaphoreType.DMA,
        ],
    )(table, idx_flat)

    out = pl.pallas_call(
        _combine_body,
        grid=(_N2P // _BC,),
        in_specs=[
            pl.BlockSpec((_BC, _K * _DP), lambda i: (i, 0)),
            pl.BlockSpec((_BC, 3), lambda i: (i, 0)),
        ],
        out_specs=pl.BlockSpec((_BC, _D), lambda i: (i, 0)),
        out_shape=jax.ShapeDtypeStruct((_N2P, _D), jnp.float32),
    )(grouped.reshape(_N2P, _K * _DP), q_pad)

    return out[:_N2], f_ref_warp


# speedup vs baseline: 7.9662x; 1.0027x over previous
import jax
import jax.numpy as jnp
from jax.experimental import pallas as pl
from jax.experimental.pallas import tpu as pltpu
from jax.experimental.pallas import tpu_sc as plsc

_N1 = 10000
_N2 = 10000
_D = 64
_K = 3
_N1P = 10240
_N2P = 10240
_BQ = 256
_BC = 400
_PAD_COORD = 1.0e6
_BIG = 3.0e38

_DP = 128
_NW = 32
_QW = _N2P // _NW
_CH = 80
_NCH = _QW // _CH


def _warp_body(c_ref, flow_ref, out_ref):
    out_ref[...] = c_ref[...] + flow_ref[...]


def _knn_body(q_ref, srcT_ref, idx_ref):
    q = q_ref[...]
    srcT = srcT_ref[...]

    sx = srcT[0:1, :]
    sy = srcT[1:2, :]
    sz = srcT[2:3, :]
    src_sq = (sx * sx + sy * sy) + sz * sz

    qx = q[:, 0:1]
    qy = q[:, 1:2]
    qz = q[:, 2:3]
    q_sq = (qx * qx + qy * qy) + qz * qz

    dot = jnp.dot(q, srcT, preferred_element_type=jnp.float32)
    d2 = (q_sq + src_sq) - 2.0 * dot

    lane = jax.lax.broadcasted_iota(jnp.int32, (_BQ, _N1P), 1)

    idxs = []
    for _ in range(_K):
        idx = jnp.argmin(d2, axis=1)[:, None].astype(jnp.int32)
        d2 = jnp.where(lane == idx, _BIG, d2)
        idxs.append(idx)

    idx_ref[...] = jnp.concatenate(idxs, axis=1)


def _gather_body(idx0_hbm, idx1_hbm, idx2_hbm, table_hbm, out_hbm,
                 idx_v, rows_v, sem):
    wid = jax.lax.axis_index("s") * 2 + jax.lax.axis_index("c")
    qbase = wid * _QW
    for k, idxk in enumerate((idx0_hbm, idx1_hbm, idx2_hbm)):
        pltpu.sync_copy(idxk.at[pl.ds(qbase, _QW)],
                        idx_v.at[pl.ds(k * _QW, _QW)])
    copies = []
    for j in range(_K * _NCH):
        copies.append(pltpu.async_copy(
            table_hbm.at[idx_v.at[pl.ds(j * _CH, _CH)]],
            rows_v.at[pl.ds(j * _CH, _CH)], sem))
    for k in range(_K):
        for c in range(_NCH):
            j = k * _NCH + c
            copies[j].wait()
            pltpu.sync_copy(rows_v.at[pl.ds(j * _CH, _CH)],
                            out_hbm.at[pl.ds(qbase + c * _CH, _CH),
                                       pl.ds(k * _DP, _DP)])


def _combine_body(g_ref, q_ref, out_ref):
    g = g_ref[...]
    q = q_ref[...]
    qx = q[:, 0:1]
    qy = q[:, 1:2]
    qz = q[:, 2:3]
    rs = []
    for k in range(_K):
        dx = g[:, k * _DP + _D:k * _DP + _D + 1] - qx
        dy = g[:, k * _DP + _D + 1:k * _DP + _D + 2] - qy
        dz = g[:, k * _DP + _D + 2:k * _DP + _D + 3] - qz
        dist = jnp.sqrt((dx * dx + dy * dy) + dz * dz)
        rs.append(1.0 / jnp.maximum(dist, 1e-10))
    norm = (rs[0] + rs[1]) + rs[2]
    out_ref[...] = (g[:, 0:_D] * (rs[0] / norm)
                    + g[:, _DP:_DP + _D] * (rs[1] / norm)) \
        + g[:, 2 * _DP:2 * _DP + _D] * (rs[2] / norm)


def kernel(f_ref_C, f_ref_F, f_cur_C, point_flow):
    f_ref_warp = pl.pallas_call(
        _warp_body,
        grid=(5,),
        in_specs=[
            pl.BlockSpec((_N1 // 5, 3), lambda i: (i, 0)),
            pl.BlockSpec((_N1 // 5, 3), lambda i: (i, 0)),
        ],
        out_specs=pl.BlockSpec((_N1 // 5, 3), lambda i: (i, 0)),
        out_shape=jax.ShapeDtypeStruct((_N1, 3), jnp.float32),
    )(f_ref_C, point_flow)

    srcT = jnp.pad(f_ref_warp, ((0, _N1P - _N1), (0, 0)),
                   constant_values=_PAD_COORD).T
    q_pad = jnp.pad(f_cur_C, ((0, _N2P - _N2), (0, 0)))
    table = jnp.pad(jnp.concatenate([f_ref_F, f_ref_warp], axis=1),
                    ((0, 0), (0, _DP - _D - 3)))

    knn_idx = pl.pallas_call(
        _knn_body,
        grid=(_N2P // _BQ,),
        in_specs=[
            pl.BlockSpec((_BQ, 3), lambda i: (i, 0)),
            pl.BlockSpec((3, _N1P), lambda i: (0, 0)),
        ],
        out_specs=pl.BlockSpec((_BQ, _K), lambda i: (i, 0)),
        out_shape=jax.ShapeDtypeStruct((_N2P, _K), jnp.int32),
    )(q_pad, srcT)

    mesh = plsc.VectorSubcoreMesh(core_axis_name="c", subcore_axis_name="s")
    grouped = pl.kernel(
        _gather_body,
        mesh=mesh,
        out_type=jax.ShapeDtypeStruct((_N2P, _K * _DP), jnp.float32),
        scratch_types=[
            pltpu.VMEM((_K * _QW,), jnp.int32),
            pltpu.VMEM((_K * _QW, _DP), jnp.float32),
            pltpu.SemaphoreType.DMA,
        ],
    )(knn_idx[:, 0], knn_idx[:, 1], knn_idx[:, 2], table)

    out = pl.pallas_call(
        _combine_body,
        grid=(_N2 // _BC,),
        in_specs=[
            pl.BlockSpec((_BC, _K * _DP), lambda i: (i, 0)),
            pl.BlockSpec((_BC, 3), lambda i: (i, 0)),
        ],
        out_specs=pl.BlockSpec((_BC, _D), lambda i: (i, 0)),
        out_shape=jax.ShapeDtypeStruct((_N2, _D), jnp.float32),
    )(grouped, f_cur_C)

    return out, f_ref_warp
